# baseline (speedup 1.0000x reference)
import functools

import jax
import jax.numpy as jnp
from jax import lax
from jax.experimental import pallas as pl
from jax.experimental.pallas import tpu as pltpu
from jax.experimental.pallas import tpu_sc as plsc

_NC = 2
_NS = 16
_NW = _NC * _NS
_LW = 128


def _gather_mean(emb, idx3, ctx, rows_per_w, chunks):
  v, d = emb.shape
  b = _NW * rows_per_w
  per_w = chunks * _LW
  mesh = plsc.VectorSubcoreMesh(core_axis_name="c", subcore_axis_name="s")

  @functools.partial(
      pl.kernel,
      mesh=mesh,
      compiler_params=pltpu.CompilerParams(use_tc_tiling_on_sc=False),
      out_type=jax.ShapeDtypeStruct((b, d), jnp.float32),
      scratch_types=[
          pltpu.VMEM((chunks, _LW), jnp.int32),
          pltpu.VMEM((per_w, d), jnp.float32),
          pltpu.VMEM((rows_per_w, d), jnp.float32),
          pltpu.SemaphoreType.DMA,
      ],
  )
  def body(emb_hbm, idx_hbm, out_hbm, idx_v, rows_v, hid_v, sem):
    wid = lax.axis_index("s") * _NC + lax.axis_index("c")
    pltpu.sync_copy(idx_hbm.at[wid], idx_v)
    for j in range(chunks):
      pltpu.async_copy(emb_hbm.at[idx_v.at[j]],
                       rows_v.at[pl.ds(j * _LW, _LW)], sem)
    for j in range(chunks):
      pltpu.make_async_copy(emb_hbm.at[idx_v.at[j]],
                            rows_v.at[pl.ds(j * _LW, _LW)], sem).wait()
    inv = jnp.float32(1.0 / ctx)

    def row_body(r, carry):
      base = r * ctx
      acc = rows_v[base, :]
      for j in range(1, ctx):
        acc = acc + rows_v[base + j, :]
      hid_v[r, :] = acc * inv
      return carry

    lax.fori_loop(0, rows_per_w, row_body, 0)
    pltpu.sync_copy(hid_v, out_hbm.at[pl.ds(wid * rows_per_w, rows_per_w)])

  return body(emb, idx3)


def _fused_body(nv, nvalid_last, wt_ref, h_ref, o_ref, m_sc, s_sc):
  p = pl.program_id(0)
  j = pl.program_id(1)

  lg = lax.dot_general(wt_ref[...], h_ref[...], (((0,), (0,)), ((), ())),
                       preferred_element_type=jnp.float32)

  @pl.when(p == 0)
  def _stats():
    @pl.when(j == 0)
    def _init():
      m_sc[...] = jnp.full(m_sc.shape, -jnp.inf, m_sc.dtype)
      s_sc[...] = jnp.zeros(s_sc.shape, s_sc.dtype)

    def upd(lgx):
      mp = jnp.max(lgx, axis=0, keepdims=True)
      m_old = m_sc[0:1, :]
      m_new = jnp.maximum(m_old, mp)
      s_new = (s_sc[0:1, :] * jnp.exp(m_old - m_new)
               + jnp.sum(jnp.exp(lgx - m_new), axis=0, keepdims=True))
      m_sc[0:1, :] = m_new
      s_sc[0:1, :] = s_new
      return m_new, s_new

    @pl.when(j < nv - 1)
    def _plain():
      upd(lg)

    @pl.when(j == nv - 1)
    def _last():
      row = lax.broadcasted_iota(jnp.int32, lg.shape, 0)
      m_new, s_new = upd(jnp.where(row < nvalid_last, lg, -jnp.inf))
      m_sc[0:1, :] = m_new + jnp.log(s_new)

  @pl.when(p == 1)
  def _write():
    o_ref[...] = lg - m_sc[0:1, :]


def _stats_body(nv, nvalid_last, wt_ref, h_ref, lse_ref, m_sc, s_sc):
  j = pl.program_id(0)

  @pl.when(j == 0)
  def _init():
    m_sc[...] = jnp.full(m_sc.shape, -jnp.inf, m_sc.dtype)
    s_sc[...] = jnp.zeros(s_sc.shape, s_sc.dtype)

  lg = lax.dot_general(wt_ref[...], h_ref[...], (((0,), (0,)), ((), ())),
                       preferred_element_type=jnp.float32)

  def upd(lgx):
    mp = jnp.max(lgx, axis=0, keepdims=True)
    m_old = m_sc[0:1, :]
    m_new = jnp.maximum(m_old, mp)
    s_new = (s_sc[0:1, :] * jnp.exp(m_old - m_new)
             + jnp.sum(jnp.exp(lgx - m_new), axis=0, keepdims=True))
    m_sc[0:1, :] = m_new
    s_sc[0:1, :] = s_new
    return m_new, s_new

  @pl.when(j < nv - 1)
  def _plain():
    upd(lg)

  @pl.when(j == nv - 1)
  def _last():
    row = lax.broadcasted_iota(jnp.int32, lg.shape, 0)
    m_new, s_new = upd(jnp.where(row < nvalid_last, lg, -jnp.inf))
    lse_ref[...] = jnp.broadcast_to(m_new + jnp.log(s_new), lse_ref.shape)


def _write_body(wt_ref, h_ref, lse_ref, o_ref):
  lg = lax.dot_general(wt_ref[...], h_ref[...], (((0,), (0,)), ((), ())),
                       preferred_element_type=jnp.float32)
  o_ref[...] = lg - lse_ref[0:1, :]


def kernel(inputs, emb, W_out):
  b, ctx = inputs.shape
  v, d = emb.shape
  total = b * ctx
  per_w = total // _NW
  chunks = per_w // _LW
  rows_per_w = b // _NW

  idx3 = inputs.astype(jnp.int32).reshape(_NW, chunks, _LW)
  hidden = _gather_mean(emb, idx3, ctx, rows_per_w, chunks)
  h_bf = hidden.astype(jnp.bfloat16)
  h_t = h_bf.T
  wt = W_out.T.astype(jnp.bfloat16)

  v_blk = 2048
  nv = pl.cdiv(v, v_blk)
  nvalid_last = v - (nv - 1) * v_blk
  out_t = pl.pallas_call(
      functools.partial(_fused_body, nv, nvalid_last),
      grid=(2, nv),
      in_specs=[
          pl.BlockSpec((d, v_blk), lambda p, j: (0, j)),
          pl.BlockSpec((d, b), lambda p, j: (0, 0)),
      ],
      out_specs=pl.BlockSpec((v_blk, b),
                             lambda p, j: (jnp.where(p == 1, j, 0), 0)),
      out_shape=jax.ShapeDtypeStruct((v, b), jnp.float32),
      scratch_shapes=[
          pltpu.VMEM((8, b), jnp.float32),
          pltpu.VMEM((8, b), jnp.float32),
      ],
  )(wt, h_t)
  return out_t.T


# speedup vs baseline: 1.0272x; 1.0272x over previous
import functools

import jax
import jax.numpy as jnp
from jax import lax
from jax.experimental import pallas as pl
from jax.experimental.pallas import tpu as pltpu
from jax.experimental.pallas import tpu_sc as plsc

_NC = 2
_NS = 16
_NW = _NC * _NS
_LW = 128


def _gather_mean(emb, idx3, ctx, rows_per_w, chunks):
  v, d = emb.shape
  b = _NW * rows_per_w
  per_w = chunks * _LW
  mesh = plsc.VectorSubcoreMesh(core_axis_name="c", subcore_axis_name="s")

  @functools.partial(
      pl.kernel,
      mesh=mesh,
      compiler_params=pltpu.CompilerParams(use_tc_tiling_on_sc=False),
      out_type=jax.ShapeDtypeStruct((b, d), jnp.float32),
      scratch_types=[
          pltpu.VMEM((chunks, _LW), jnp.int32),
          pltpu.VMEM((per_w, d), jnp.float32),
          pltpu.VMEM((rows_per_w, d), jnp.float32),
          pltpu.SemaphoreType.DMA,
      ],
  )
  def body(emb_hbm, idx_hbm, out_hbm, idx_v, rows_v, hid_v, sem):
    wid = lax.axis_index("s") * _NC + lax.axis_index("c")
    pltpu.sync_copy(idx_hbm.at[wid], idx_v)
    for j in range(chunks):
      pltpu.async_copy(emb_hbm.at[idx_v.at[j]],
                       rows_v.at[pl.ds(j * _LW, _LW)], sem)
    for j in range(chunks):
      pltpu.make_async_copy(emb_hbm.at[idx_v.at[j]],
                            rows_v.at[pl.ds(j * _LW, _LW)], sem).wait()
    inv = jnp.float32(1.0 / ctx)

    def row_body(r, carry):
      base = r * ctx
      acc = rows_v[base, :]
      for j in range(1, ctx):
        acc = acc + rows_v[base + j, :]
      hid_v[r, :] = acc * inv
      return carry

    lax.fori_loop(0, rows_per_w, row_body, 0)
    pltpu.sync_copy(hid_v, out_hbm.at[pl.ds(wid * rows_per_w, rows_per_w)])

  return body(emb, idx3)


def _fused_body(nv, nvalid_last, wt_ref, h_ref, o_ref, m_sc, s_sc):
  p = pl.program_id(0)
  j = pl.program_id(1)

  lg = lax.dot_general(wt_ref[...], h_ref[...], (((0,), (0,)), ((), ())),
                       preferred_element_type=jnp.float32)

  @pl.when(p == 0)
  def _stats():
    @pl.when(j == 0)
    def _init():
      m_sc[...] = jnp.full(m_sc.shape, -jnp.inf, m_sc.dtype)
      s_sc[...] = jnp.zeros(s_sc.shape, s_sc.dtype)

    def upd(lgx):
      mp = jnp.max(lgx, axis=0, keepdims=True)
      m_old = m_sc[0:1, :]
      m_new = jnp.maximum(m_old, mp)
      s_new = (s_sc[0:1, :] * jnp.exp(m_old - m_new)
               + jnp.sum(jnp.exp(lgx - m_new), axis=0, keepdims=True))
      m_sc[0:1, :] = m_new
      s_sc[0:1, :] = s_new
      return m_new, s_new

    @pl.when(j < nv - 1)
    def _plain():
      upd(lg)

    @pl.when(j == nv - 1)
    def _last():
      row = lax.broadcasted_iota(jnp.int32, lg.shape, 0)
      m_new, s_new = upd(jnp.where(row < nvalid_last, lg, -jnp.inf))
      m_sc[0:1, :] = m_new + jnp.log(s_new)

  @pl.when(p == 1)
  def _write():
    o_ref[...] = lg - m_sc[0:1, :]


def _stats_body(nv, nvalid_last, wt_ref, h_ref, lse_ref, m_sc, s_sc):
  j = pl.program_id(0)

  @pl.when(j == 0)
  def _init():
    m_sc[...] = jnp.full(m_sc.shape, -jnp.inf, m_sc.dtype)
    s_sc[...] = jnp.zeros(s_sc.shape, s_sc.dtype)

  lg = lax.dot_general(wt_ref[...], h_ref[...], (((0,), (0,)), ((), ())),
                       preferred_element_type=jnp.float32)

  def upd(lgx):
    mp = jnp.max(lgx, axis=0, keepdims=True)
    m_old = m_sc[0:1, :]
    m_new = jnp.maximum(m_old, mp)
    s_new = (s_sc[0:1, :] * jnp.exp(m_old - m_new)
             + jnp.sum(jnp.exp(lgx - m_new), axis=0, keepdims=True))
    m_sc[0:1, :] = m_new
    s_sc[0:1, :] = s_new
    return m_new, s_new

  @pl.when(j < nv - 1)
  def _plain():
    upd(lg)

  @pl.when(j == nv - 1)
  def _last():
    row = lax.broadcasted_iota(jnp.int32, lg.shape, 0)
    m_new, s_new = upd(jnp.where(row < nvalid_last, lg, -jnp.inf))
    lse_ref[...] = jnp.broadcast_to(m_new + jnp.log(s_new), lse_ref.shape)


def _write_body(wt_ref, h_ref, lse_ref, o_ref):
  lg = lax.dot_general(wt_ref[...], h_ref[...], (((0,), (0,)), ((), ())),
                       preferred_element_type=jnp.float32)
  o_ref[...] = lg - lse_ref[0:1, :]


def kernel(inputs, emb, W_out):
  b, ctx = inputs.shape
  v, d = emb.shape
  total = b * ctx
  per_w = total // _NW
  chunks = per_w // _LW
  rows_per_w = b // _NW

  idx3 = inputs.astype(jnp.int32).reshape(_NW, chunks, _LW)
  hidden = _gather_mean(emb, idx3, ctx, rows_per_w, chunks)
  h_bf = hidden.astype(jnp.bfloat16)
  h_t = h_bf.T
  wt = W_out.T.astype(jnp.bfloat16)

  v_blk = 4096
  nv = pl.cdiv(v, v_blk)
  nvalid_last = v - (nv - 1) * v_blk
  lse_row = pl.pallas_call(
      functools.partial(_stats_body, nv, nvalid_last),
      grid=(nv,),
      in_specs=[
          pl.BlockSpec((d, v_blk), lambda j: (0, j)),
          pl.BlockSpec((d, b), lambda j: (0, 0)),
      ],
      out_specs=pl.BlockSpec((8, b), lambda j: (0, 0)),
      out_shape=jax.ShapeDtypeStruct((8, b), jnp.float32),
      scratch_shapes=[
          pltpu.VMEM((8, b), jnp.float32),
          pltpu.VMEM((8, b), jnp.float32),
      ],
  )(wt, h_t)

  w_blk = 2048
  out_t = pl.pallas_call(
      _write_body,
      grid=(pl.cdiv(v, w_blk),),
      in_specs=[
          pl.BlockSpec((d, w_blk), lambda j: (0, j)),
          pl.BlockSpec((d, b), lambda j: (0, 0)),
          pl.BlockSpec((8, b), lambda j: (0, 0)),
      ],
      out_specs=pl.BlockSpec((w_blk, b), lambda j: (j, 0)),
      out_shape=jax.ShapeDtypeStruct((v, b), jnp.float32),
  )(wt, h_t, lse_row)
  return out_t.T
